# baseline (device time: 179611 ns/iter reference)
import jax
import jax.numpy as jnp
from jax import lax
from jax.experimental import pallas as pl
from jax.experimental.pallas import tpu as pltpu

N_DEV = 8
N_SUB = 2
AG_SUB = 4
AMAX_HOPS = (4, 3)


def kernel(x, w_mat):
    m, k_per = x.shape
    _, n = w_mat.shape
    ch = m // N_DEV
    half = ch // 2
    sub = n // N_SUB

    def body(x_ref, w_ref, out_ref, comm_ref, sbuf_ref, q_ref,
             amax_sbuf, amax_rbuf,
             rs_send_sems, rs_recv_sems, credit_sems,
             ax_send_sems, ax_recv_sems,
             ag_send_sems, ag_recv_sems):
        me = lax.axis_index("i")
        left = (me - 1) % N_DEV
        right = (me + 1) % N_DEV

        barrier_sem = pltpu.get_barrier_semaphore()
        for nbr in (left, right):
            pl.semaphore_signal(
                barrier_sem, inc=1,
                device_id=(nbr,), device_id_type=pl.DeviceIdType.MESH,
            )
        pl.semaphore_wait(barrier_sem, 2)

        def rows_h(c, d):
            return pl.ds(c * ch + d * half, half)

        def cols(u):
            return pl.ds(u * sub, sub)

        ag_sub = n // AG_SUB

        def ag_cols(u):
            return pl.ds(u * ag_sub, ag_sub)

        def rs_send_chunk(d, s):
            return (me - s) % N_DEV if d == 0 else (me + s) % N_DEV

        def rs_recv_chunk(d, s):
            return (me - s - 1) % N_DEV if d == 0 else (me + s + 1) % N_DEV

        def ag_send_chunk(d, t):
            return (me + 1 - t) % N_DEV if d == 0 else (me - 1 + t) % N_DEV

        def ag_recv_chunk(d, t):
            return (me - t) % N_DEV if d == 0 else (me + t) % N_DEV

        to_nbr = (lambda d: right if d == 0 else left)
        from_nbr = (lambda d: left if d == 0 else right)

        def fill_sbuf(d, u, s):
            sbuf_ref[d, u] = out_ref[
                rows_h(rs_send_chunk(d, s), d), cols(u)
            ].astype(jnp.bfloat16)

        def start_rs(d, u, s):
            rdma = pltpu.make_async_remote_copy(
                src_ref=sbuf_ref.at[d, u],
                dst_ref=comm_ref.at[d, u],
                send_sem=rs_send_sems.at[d, s, u],
                recv_sem=rs_recv_sems.at[d, s, u],
                device_id=(to_nbr(d),),
                device_id_type=pl.DeviceIdType.MESH,
            )
            rdma.start()
            return rdma

        out_ref[pl.ds(me * ch, ch), :] = jnp.dot(
            x_ref[pl.ds(me * ch, ch), :], w_ref[...],
            preferred_element_type=jnp.float32,
        )
        rdmas = {}
        for u in (0, 1):
            for d in (0, 1):
                fill_sbuf(d, u, 0)
                rdmas[d, u] = start_rs(d, u, 0)

        def gemm_body(j, carry):
            c = (me - j) % N_DEV
            out_ref[pl.ds(c * ch, ch), :] = jnp.dot(
                x_ref[pl.ds(c * ch, ch), :], w_ref[...],
                preferred_element_type=jnp.float32,
            )
            return carry

        lax.fori_loop(1, N_DEV, gemm_body, 0)

        for s in range(N_DEV - 1):
            for u in (0, 1):
                for d in (0, 1):
                    rdmas[d, u].wait()
                    out_ref[rows_h(rs_recv_chunk(d, s), d), cols(u)] += (
                        comm_ref[d, u].astype(jnp.float32)
                    )
                    if s <= N_DEV - 3:
                        pl.semaphore_signal(
                            credit_sems.at[d, u], inc=1,
                            device_id=(from_nbr(d),),
                            device_id_type=pl.DeviceIdType.MESH,
                        )
                    if s + 1 <= N_DEV - 2:
                        fill_sbuf(d, u, s + 1)
                        pl.semaphore_wait(credit_sems.at[d, u], 1)
                        rdmas[d, u] = start_rs(d, u, s + 1)

        own0 = rows_h((me + 1) % N_DEV, 0)
        own1 = rows_h((me - 1) % N_DEV, 1)
        local_amax = jnp.maximum(
            jnp.max(jnp.abs(out_ref[own0, :])),
            jnp.max(jnp.abs(out_ref[own1, :])),
        )
        r = [local_amax, local_amax]
        for h in range(max(AMAX_HOPS)):
            hop_rdmas = []
            active = [d for d in (0, 1) if h < AMAX_HOPS[d]]
            for d in active:
                amax_sbuf[d, :, :] = jnp.full((1, 128), r[d], jnp.float32)
                rdma = pltpu.make_async_remote_copy(
                    src_ref=amax_sbuf.at[d],
                    dst_ref=amax_rbuf.at[d, h],
                    send_sem=ax_send_sems.at[d, h],
                    recv_sem=ax_recv_sems.at[d, h],
                    device_id=(to_nbr(d),),
                    device_id_type=pl.DeviceIdType.MESH,
                )
                rdma.start()
                hop_rdmas.append(rdma)
            for d, rdma in zip(active, hop_rdmas):
                rdma.wait()
                r[d] = jnp.maximum(r[d], amax_rbuf[d, h, 0, 0])
        amax = jnp.maximum(r[0], r[1])
        scale = amax / 448.0
        inv_scale = 448.0 / amax

        for rows in (own0, own1):
            y = out_ref[rows, :]
            q = (y * inv_scale).astype(jnp.float8_e4m3fn)
            q_ref[rows, :] = q
            out_ref[rows, :] = q.astype(jnp.float32) * scale

        def start_ag(d, t, u):
            rc = rows_h(ag_send_chunk(d, t), d)
            rdma = pltpu.make_async_remote_copy(
                src_ref=q_ref.at[rc, ag_cols(u)],
                dst_ref=q_ref.at[rc, ag_cols(u)],
                send_sem=ag_send_sems.at[d, t, u],
                recv_sem=ag_recv_sems.at[d, t, u],
                device_id=(to_nbr(d),),
                device_id_type=pl.DeviceIdType.MESH,
            )
            rdma.start()
            return rdma

        ag_rdmas = {}
        for u in range(AG_SUB):
            for d in (0, 1):
                ag_rdmas[d, u] = start_ag(d, 0, u)
        for t in range(N_DEV - 1):
            for u in range(AG_SUB):
                for d in (0, 1):
                    ag_rdmas[d, u].wait()
                    if t + 1 <= N_DEV - 2:
                        ag_rdmas[d, u] = start_ag(d, t + 1, u)
            for d in (0, 1):
                rc = rows_h(ag_recv_chunk(d, t), d)
                out_ref[rc, :] = q_ref[rc, :].astype(jnp.float32) * scale

    return pl.pallas_call(
        body,
        out_shape=jax.ShapeDtypeStruct((m, n), jnp.float32),
        in_specs=[
            pl.BlockSpec(memory_space=pltpu.VMEM),
            pl.BlockSpec(memory_space=pltpu.VMEM),
        ],
        out_specs=pl.BlockSpec(memory_space=pltpu.VMEM),
        scratch_shapes=[
            pltpu.VMEM((2, N_SUB, half, sub), jnp.bfloat16),
            pltpu.VMEM((2, N_SUB, half, sub), jnp.bfloat16),
            pltpu.VMEM((m, n), jnp.float8_e4m3fn),
            pltpu.VMEM((2, 1, 128), jnp.float32),
            pltpu.VMEM((2, max(AMAX_HOPS), 1, 128), jnp.float32),
            pltpu.SemaphoreType.DMA((2, N_DEV - 1, N_SUB)),
            pltpu.SemaphoreType.DMA((2, N_DEV - 1, N_SUB)),
            pltpu.SemaphoreType.REGULAR((2, N_SUB)),
            pltpu.SemaphoreType.DMA((2, max(AMAX_HOPS))),
            pltpu.SemaphoreType.DMA((2, max(AMAX_HOPS))),
            pltpu.SemaphoreType.DMA((2, N_DEV - 1, AG_SUB)),
            pltpu.SemaphoreType.DMA((2, N_DEV - 1, AG_SUB)),
        ],
        compiler_params=pltpu.CompilerParams(
            collective_id=0,
            vmem_limit_bytes=60 * 1024 * 1024,
        ),
    )(x, w_mat)


# device time: 178960 ns/iter; 1.0036x vs baseline; 1.0036x over previous
import jax
import jax.numpy as jnp
from jax import lax
from jax.experimental import pallas as pl
from jax.experimental.pallas import tpu as pltpu

N_DEV = 8
N_SUB = 2
AG_SUB = 2
AMAX_HOPS = (4, 3)


def kernel(x, w_mat):
    m, k_per = x.shape
    _, n = w_mat.shape
    ch = m // N_DEV
    half = ch // 2
    sub = n // N_SUB

    def body(x_ref, w_ref, out_ref, comm_ref, sbuf_ref, q_ref,
             amax_sbuf, amax_rbuf,
             rs_send_sems, rs_recv_sems, credit_sems,
             ax_send_sems, ax_recv_sems,
             ag_send_sems, ag_recv_sems):
        me = lax.axis_index("i")
        left = (me - 1) % N_DEV
        right = (me + 1) % N_DEV

        barrier_sem = pltpu.get_barrier_semaphore()
        for nbr in (left, right):
            pl.semaphore_signal(
                barrier_sem, inc=1,
                device_id=(nbr,), device_id_type=pl.DeviceIdType.MESH,
            )
        pl.semaphore_wait(barrier_sem, 2)

        def rows_h(c, d):
            return pl.ds(c * ch + d * half, half)

        def cols(u):
            return pl.ds(u * sub, sub)

        ag_sub = n // AG_SUB

        def ag_cols(u):
            return pl.ds(u * ag_sub, ag_sub)

        def rs_send_chunk(d, s):
            return (me - s) % N_DEV if d == 0 else (me + s) % N_DEV

        def rs_recv_chunk(d, s):
            return (me - s - 1) % N_DEV if d == 0 else (me + s + 1) % N_DEV

        def ag_send_chunk(d, t):
            return (me + 1 - t) % N_DEV if d == 0 else (me - 1 + t) % N_DEV

        def ag_recv_chunk(d, t):
            return (me - t) % N_DEV if d == 0 else (me + t) % N_DEV

        to_nbr = (lambda d: right if d == 0 else left)
        from_nbr = (lambda d: left if d == 0 else right)

        def fill_sbuf(d, u, s):
            sbuf_ref[d, u] = out_ref[
                rows_h(rs_send_chunk(d, s), d), cols(u)
            ].astype(jnp.bfloat16)

        def start_rs(d, u, s):
            rdma = pltpu.make_async_remote_copy(
                src_ref=sbuf_ref.at[d, u],
                dst_ref=comm_ref.at[d, u],
                send_sem=rs_send_sems.at[d, s, u],
                recv_sem=rs_recv_sems.at[d, s, u],
                device_id=(to_nbr(d),),
                device_id_type=pl.DeviceIdType.MESH,
            )
            rdma.start()
            return rdma

        out_ref[pl.ds(me * ch, ch), :] = jnp.dot(
            x_ref[pl.ds(me * ch, ch), :], w_ref[...],
            preferred_element_type=jnp.float32,
        )
        rdmas = {}
        for u in (0, 1):
            for d in (0, 1):
                fill_sbuf(d, u, 0)
                rdmas[d, u] = start_rs(d, u, 0)

        def gemm_body(j, carry):
            c = (me - j) % N_DEV
            out_ref[pl.ds(c * ch, ch), :] = jnp.dot(
                x_ref[pl.ds(c * ch, ch), :], w_ref[...],
                preferred_element_type=jnp.float32,
            )
            return carry

        lax.fori_loop(1, N_DEV, gemm_body, 0)

        for s in range(N_DEV - 1):
            for u in (0, 1):
                for d in (0, 1):
                    rdmas[d, u].wait()
                    out_ref[rows_h(rs_recv_chunk(d, s), d), cols(u)] += (
                        comm_ref[d, u].astype(jnp.float32)
                    )
                    if s <= N_DEV - 3:
                        pl.semaphore_signal(
                            credit_sems.at[d, u], inc=1,
                            device_id=(from_nbr(d),),
                            device_id_type=pl.DeviceIdType.MESH,
                        )
                    if s + 1 <= N_DEV - 2:
                        fill_sbuf(d, u, s + 1)
                        pl.semaphore_wait(credit_sems.at[d, u], 1)
                        rdmas[d, u] = start_rs(d, u, s + 1)

        own0 = rows_h((me + 1) % N_DEV, 0)
        own1 = rows_h((me - 1) % N_DEV, 1)
        local_amax = jnp.maximum(
            jnp.max(jnp.abs(out_ref[own0, :])),
            jnp.max(jnp.abs(out_ref[own1, :])),
        )
        r = [local_amax, local_amax]
        for h in range(max(AMAX_HOPS)):
            hop_rdmas = []
            active = [d for d in (0, 1) if h < AMAX_HOPS[d]]
            for d in active:
                amax_sbuf[d, :, :] = jnp.full((1, 128), r[d], jnp.float32)
                rdma = pltpu.make_async_remote_copy(
                    src_ref=amax_sbuf.at[d],
                    dst_ref=amax_rbuf.at[d, h],
                    send_sem=ax_send_sems.at[d, h],
                    recv_sem=ax_recv_sems.at[d, h],
                    device_id=(to_nbr(d),),
                    device_id_type=pl.DeviceIdType.MESH,
                )
                rdma.start()
                hop_rdmas.append(rdma)
            for d, rdma in zip(active, hop_rdmas):
                rdma.wait()
                r[d] = jnp.maximum(r[d], amax_rbuf[d, h, 0, 0])
        amax = jnp.maximum(r[0], r[1])
        scale = amax / 448.0
        inv_scale = 448.0 / amax

        for rows in (own0, own1):
            y = out_ref[rows, :]
            q = (y * inv_scale).astype(jnp.float8_e4m3fn)
            q_ref[rows, :] = q
            out_ref[rows, :] = q.astype(jnp.float32) * scale

        def start_ag(d, t, u):
            rc = rows_h(ag_send_chunk(d, t), d)
            rdma = pltpu.make_async_remote_copy(
                src_ref=q_ref.at[rc, ag_cols(u)],
                dst_ref=q_ref.at[rc, ag_cols(u)],
                send_sem=ag_send_sems.at[d, t, u],
                recv_sem=ag_recv_sems.at[d, t, u],
                device_id=(to_nbr(d),),
                device_id_type=pl.DeviceIdType.MESH,
            )
            rdma.start()
            return rdma

        ag_rdmas = {}
        for u in range(AG_SUB):
            for d in (0, 1):
                ag_rdmas[d, u] = start_ag(d, 0, u)
        for t in range(N_DEV - 1):
            for u in range(AG_SUB):
                for d in (0, 1):
                    ag_rdmas[d, u].wait()
                    if t + 1 <= N_DEV - 2:
                        ag_rdmas[d, u] = start_ag(d, t + 1, u)
            for d in (0, 1):
                rc = rows_h(ag_recv_chunk(d, t), d)
                out_ref[rc, :] = q_ref[rc, :].astype(jnp.float32) * scale

    return pl.pallas_call(
        body,
        out_shape=jax.ShapeDtypeStruct((m, n), jnp.float32),
        in_specs=[
            pl.BlockSpec(memory_space=pltpu.VMEM),
            pl.BlockSpec(memory_space=pltpu.VMEM),
        ],
        out_specs=pl.BlockSpec(memory_space=pltpu.VMEM),
        scratch_shapes=[
            pltpu.VMEM((2, N_SUB, half, sub), jnp.bfloat16),
            pltpu.VMEM((2, N_SUB, half, sub), jnp.bfloat16),
            pltpu.VMEM((m, n), jnp.float8_e4m3fn),
            pltpu.VMEM((2, 1, 128), jnp.float32),
            pltpu.VMEM((2, max(AMAX_HOPS), 1, 128), jnp.float32),
            pltpu.SemaphoreType.DMA((2, N_DEV - 1, N_SUB)),
            pltpu.SemaphoreType.DMA((2, N_DEV - 1, N_SUB)),
            pltpu.SemaphoreType.REGULAR((2, N_SUB)),
            pltpu.SemaphoreType.DMA((2, max(AMAX_HOPS))),
            pltpu.SemaphoreType.DMA((2, max(AMAX_HOPS))),
            pltpu.SemaphoreType.DMA((2, N_DEV - 1, AG_SUB)),
            pltpu.SemaphoreType.DMA((2, N_DEV - 1, AG_SUB)),
        ],
        compiler_params=pltpu.CompilerParams(
            collective_id=0,
            vmem_limit_bytes=60 * 1024 * 1024,
        ),
    )(x, w_mat)
